# baseline (speedup 1.0000x reference)
import jax
import jax.numpy as jnp
from jax import lax
from jax.experimental import pallas as pl
from jax.experimental.pallas import tpu as pltpu
from jax.experimental.pallas import tpu_sc as plsc

NUM_CLAUSES_P1 = 9
HIDDEN = 2048
LANES = 16
NGRP = HIDDEN // LANES
BATCH = 16384

_INFO = plsc.get_sparse_core_info()
NC = _INFO.num_cores
NS = _INFO.num_subcores
NW = NC * NS
B_PER_W = BATCH // NW
CHUNK = 16
NCHUNK = B_PER_W // CHUNK
NBUF = 2


def _sc_body(idx_hbm, table_hbm, out_hbm, table_v, idx_s,
             buf0, buf1, ss0, ss1):
    bufs = (buf0, buf1)
    ssems = (ss0, ss1)
    cid = lax.axis_index("c")
    sid = lax.axis_index("s")
    wid = sid * NC + cid
    base = wid * B_PER_W

    pltpu.sync_copy(table_hbm, table_v)
    pltpu.sync_copy(idx_hbm.at[pl.ds(base, B_PER_W)], idx_s)

    lane_iota = lax.iota(jnp.int32, LANES)
    step16 = jnp.full((LANES,), LANES, jnp.int32)

    def fill(c, b):
        ivec = idx_s[pl.ds(c * CHUNK, LANES)] * HIDDEN
        for r in range(CHUNK):
            row_base = jnp.broadcast_to(ivec[r], (LANES,)) + lane_iota

            def grp_step(g, idxv, r=r):
                bufs[b][r, pl.ds(g * LANES, LANES)] = plsc.load_gather(
                    table_v, [idxv])
                return idxv + step16

            lax.fori_loop(0, NGRP, grp_step, row_base, unroll=16)

    def store(c, b):
        return pltpu.make_async_copy(
            bufs[b], out_hbm.at[pl.ds(base + c * CHUNK, CHUNK)], ssems[b])

    for b in range(NBUF):
        fill(b, b)
        store(b, b).start()

    def step(c, carry):
        for bb in range(NBUF):
            @pl.when(lax.rem(c, NBUF) == bb)
            def _(bb=bb):
                store(c - NBUF, bb).wait()
                fill(c, bb)
                store(c, bb).start()
        return carry

    lax.fori_loop(NBUF, NCHUNK, step, 0)

    for b in range(NBUF):
        store(NCHUNK - NBUF + b, (NCHUNK - NBUF + b) % NBUF).wait()


@jax.jit
def kernel(clause_indices, embeddings):
    idx = clause_indices.astype(jnp.int32)
    table_flat = embeddings.reshape(NUM_CLAUSES_P1 * HIDDEN)
    mesh = plsc.VectorSubcoreMesh(core_axis_name="c", subcore_axis_name="s")
    f = pl.kernel(
        _sc_body,
        out_type=jax.ShapeDtypeStruct((BATCH, HIDDEN), jnp.float32),
        mesh=mesh,
        compiler_params=pltpu.CompilerParams(needs_layout_passes=False),
        scratch_types=[
            pltpu.VMEM((NUM_CLAUSES_P1 * HIDDEN,), jnp.float32),
            pltpu.VMEM((B_PER_W,), jnp.int32),
            pltpu.VMEM((CHUNK, HIDDEN), jnp.float32),
            pltpu.VMEM((CHUNK, HIDDEN), jnp.float32),
            pltpu.SemaphoreType.DMA,
            pltpu.SemaphoreType.DMA,
        ],
    )
    return f(idx, table_flat)


# speedup vs baseline: 1.4785x; 1.4785x over previous
import jax
import jax.numpy as jnp
from jax import lax
from jax.experimental import pallas as pl
from jax.experimental.pallas import tpu as pltpu
from jax.experimental.pallas import tpu_sc as plsc

NUM_CLAUSES_P1 = 9
HIDDEN = 2048
LANES = 16
NGRP = HIDDEN // LANES
BATCH = 16384

_INFO = plsc.get_sparse_core_info()
NC = _INFO.num_cores
NS = _INFO.num_subcores
NW = NC * NS
B_PER_W = BATCH // NW
CHUNK = 16
NCHUNK = B_PER_W // CHUNK
NBUF = 2


def _sc_body(idx_hbm, table_hbm, out_hbm, table_v, idx_s,
             buf0, buf1, ss0, ss1):
    bufs = (buf0, buf1)
    ssems = (ss0, ss1)
    cid = lax.axis_index("c")
    sid = lax.axis_index("s")
    wid = sid * NC + cid
    base = wid * B_PER_W

    pltpu.sync_copy(table_hbm, table_v)
    pltpu.sync_copy(idx_hbm.at[pl.ds(base, B_PER_W)], idx_s)

    lane_iota = lax.iota(jnp.int32, LANES)
    step16 = jnp.full((LANES,), LANES, jnp.int32)

    def fill(c, b):
        ivec = idx_s[pl.ds(c * CHUNK, LANES)] * HIDDEN
        for r in range(CHUNK):
            row_base = jnp.broadcast_to(ivec[r], (LANES,)) + lane_iota

            @plsc.parallel_loop(0, NGRP, 1, unroll=16, carry=row_base)
            def _(g, idxv, r=r):
                bufs[b][pl.ds(r * HIDDEN + g * LANES, LANES)] = (
                    plsc.load_gather(table_v, [idxv]))
                return idxv + step16

    def store(c, b):
        return pltpu.make_async_copy(
            bufs[b],
            out_hbm.at[pl.ds((base + c * CHUNK) * HIDDEN, CHUNK * HIDDEN)],
            ssems[b])

    for b in range(NBUF):
        fill(b, b)
        store(b, b).start()

    def step(c, carry):
        for bb in range(NBUF):
            @pl.when(lax.rem(c, NBUF) == bb)
            def _(bb=bb):
                store(c - NBUF, bb).wait()
                fill(c, bb)
                store(c, bb).start()
        return carry

    lax.fori_loop(NBUF, NCHUNK, step, 0)

    for b in range(NBUF):
        store(NCHUNK - NBUF + b, (NCHUNK - NBUF + b) % NBUF).wait()


@jax.jit
def kernel(clause_indices, embeddings):
    idx = clause_indices.astype(jnp.int32)
    table_flat = embeddings.reshape(NUM_CLAUSES_P1 * HIDDEN)
    mesh = plsc.VectorSubcoreMesh(core_axis_name="c", subcore_axis_name="s")
    f = pl.kernel(
        _sc_body,
        out_type=jax.ShapeDtypeStruct((BATCH * HIDDEN,), jnp.float32),
        mesh=mesh,
        compiler_params=pltpu.CompilerParams(needs_layout_passes=False),
        scratch_types=[
            pltpu.VMEM((NUM_CLAUSES_P1 * HIDDEN,), jnp.float32),
            pltpu.VMEM((B_PER_W,), jnp.int32),
            pltpu.VMEM((CHUNK * HIDDEN,), jnp.float32),
            pltpu.VMEM((CHUNK * HIDDEN,), jnp.float32),
            pltpu.SemaphoreType.DMA,
            pltpu.SemaphoreType.DMA,
        ],
    )
    return f(idx, table_flat).reshape(BATCH, HIDDEN)


# speedup vs baseline: 1.5464x; 1.0460x over previous
import jax
import jax.numpy as jnp
from jax import lax
from jax.experimental import pallas as pl
from jax.experimental.pallas import tpu as pltpu
from jax.experimental.pallas import tpu_sc as plsc

NUM_CLAUSES_P1 = 9
HIDDEN = 2048
LANES = 16
BATCH = 16384

_INFO = plsc.get_sparse_core_info()
NC = _INFO.num_cores
NS = _INFO.num_subcores
NW = NC * NS
B_PER_W = BATCH // NW
NGROUPS = B_PER_W // LANES


def _sc_body(idx_hbm, table_hbm, out_hbm, table_v, idx_s, sem):
    cid = lax.axis_index("c")
    sid = lax.axis_index("s")
    wid = sid * NC + cid
    base = wid * B_PER_W

    pltpu.sync_copy(table_hbm, table_v)
    pltpu.sync_copy(idx_hbm.at[pl.ds(base, B_PER_W)], idx_s)

    def group_step(rg, carry):
        ivec = idx_s[pl.ds(rg * LANES, LANES)] * HIDDEN
        row0 = (base + rg * LANES) * HIDDEN
        for l in range(LANES):
            pltpu.make_async_copy(
                table_v.at[pl.ds(pl.multiple_of(ivec[l], HIDDEN), HIDDEN)],
                out_hbm.at[pl.ds(pl.multiple_of(row0 + l * HIDDEN, HIDDEN),
                                 HIDDEN)],
                sem,
            ).start()
        return carry

    lax.fori_loop(0, NGROUPS, group_step, 0)

    def drain_step(r, carry):
        pltpu.make_async_copy(
            table_v.at[pl.ds(0, HIDDEN)],
            out_hbm.at[pl.ds(pl.multiple_of(base * HIDDEN, HIDDEN), HIDDEN)],
            sem,
        ).wait()
        return carry

    lax.fori_loop(0, B_PER_W, drain_step, 0)


@jax.jit
def kernel(clause_indices, embeddings):
    idx = clause_indices.astype(jnp.int32)
    table_flat = embeddings.reshape(NUM_CLAUSES_P1 * HIDDEN)
    mesh = plsc.VectorSubcoreMesh(core_axis_name="c", subcore_axis_name="s")
    f = pl.kernel(
        _sc_body,
        out_type=jax.ShapeDtypeStruct((BATCH * HIDDEN,), jnp.float32),
        mesh=mesh,
        compiler_params=pltpu.CompilerParams(needs_layout_passes=False),
        scratch_types=[
            pltpu.VMEM((NUM_CLAUSES_P1 * HIDDEN,), jnp.float32),
            pltpu.VMEM((B_PER_W,), jnp.int32),
            pltpu.SemaphoreType.DMA,
        ],
    )
    return f(idx, table_flat).reshape(BATCH, HIDDEN)
